# Initial kernel scaffold; baseline (speedup 1.0000x reference)
import jax
import jax.numpy as jnp
from jax.experimental import pallas as pl


def kernel(x, edge_index, edge_attr, enn_w1, enn_b1, enn_w2, enn_b2, root_w, nnconv_b, gat0_lin, gat0_asrc, gat0_adst, gat0_b, gat1_lin, gat1_asrc, gat1_adst, gat1_b, gat2_lin, gat2_asrc, gat2_adst, gat2_b, gat3_lin, gat3_asrc, gat3_adst, gat3_b, mlp0_w, mlp0_b, mlp1_w, mlp1_b, mlp2_w, mlp2_b):
    raise NotImplementedError("write your pallas kernel here")


# speedup vs baseline: 14.5788x; 14.5788x over previous
import functools
import jax
import jax.numpy as jnp
from jax import lax
from jax.experimental import pallas as pl
from jax.experimental.pallas import tpu as pltpu
from jax.experimental.pallas import tpu_sc as plsc

N = 10000
E = 160000
E_PAD = 163840
IN = 128
EMB = 8
H = 8
HID = 32
NEG = 0.2

NW = 32
BE = 1024
BN = 2000
F32 = jnp.float32

_MESH = plsc.VectorSubcoreMesh(core_axis_name="c", subcore_axis_name="s")


def _make_gather2(ca, cb):
    epw = E_PAD // NW
    ch = 128
    nch = epw // ch

    @functools.partial(
        pl.kernel, mesh=_MESH,
        out_type=(jax.ShapeDtypeStruct((E_PAD, ca), F32),
                  jax.ShapeDtypeStruct((E_PAD, cb), F32)),
        scratch_types=[pltpu.VMEM((ch,), jnp.int32), pltpu.VMEM((ch,), jnp.int32),
                       pltpu.VMEM((ch, ca), F32), pltpu.VMEM((ch, cb), F32),
                       pltpu.SemaphoreType.DMA, pltpu.SemaphoreType.DMA],
    )
    def gk(ia_hbm, ta_hbm, ib_hbm, tb_hbm, oa_hbm, ob_hbm,
           ia_v, ib_v, ra_v, rb_v, sa, sb):
        wid = lax.axis_index("s") * 2 + lax.axis_index("c")
        base = wid * epw

        def body(j, carry):
            off = pl.multiple_of(base + j * ch, ch)
            pltpu.sync_copy(ia_hbm.at[pl.ds(off, ch)], ia_v)
            pltpu.sync_copy(ib_hbm.at[pl.ds(off, ch)], ib_v)
            da = pltpu.async_copy(ta_hbm.at[ia_v], ra_v, sa)
            db = pltpu.async_copy(tb_hbm.at[ib_v], rb_v, sb)
            da.wait()
            db.wait()
            pltpu.sync_copy(ra_v, oa_hbm.at[pl.ds(off, ch)])
            pltpu.sync_copy(rb_v, ob_hbm.at[pl.ds(off, ch)])
            return carry

        lax.fori_loop(0, nch, body, 0)

    return gk


def _make_gather1(ca):
    epw = E_PAD // NW
    ch = 128
    nch = epw // ch

    @functools.partial(
        pl.kernel, mesh=_MESH,
        out_type=jax.ShapeDtypeStruct((E_PAD, ca), F32),
        scratch_types=[pltpu.VMEM((ch,), jnp.int32),
                       pltpu.VMEM((ch, ca), F32),
                       pltpu.SemaphoreType.DMA],
    )
    def gk(ia_hbm, ta_hbm, oa_hbm, ia_v, ra_v, sa):
        wid = lax.axis_index("s") * 2 + lax.axis_index("c")
        base = wid * epw

        def body(j, carry):
            off = pl.multiple_of(base + j * ch, ch)
            pltpu.sync_copy(ia_hbm.at[pl.ds(off, ch)], ia_v)
            pltpu.async_copy(ta_hbm.at[ia_v], ra_v, sa).wait()
            pltpu.sync_copy(ra_v, oa_hbm.at[pl.ds(off, ch)])
            return carry

        lax.fori_loop(0, nch, body, 0)

    return gk


def _make_scatter_cols():
    ept = E_PAD // 16
    ch = 128
    nch = ept // ch
    cs = 128
    rpt = 640

    @functools.partial(
        pl.kernel, mesh=_MESH,
        out_type=jax.ShapeDtypeStruct((N, 256), F32),
        scratch_types=[pltpu.VMEM((ch,), jnp.int32),
                       pltpu.VMEM((ch, cs), F32),
                       pltpu.VMEM_SHARED((N, cs), F32)],
    )
    def sk(idx_hbm, vals_hbm, zer_hbm, out_hbm, idx_v, val_v, sp):
        cid = lax.axis_index("c")
        sid = lax.axis_index("s")
        col0 = cid * cs

        @pl.when(sid == 0)
        def _():
            pltpu.sync_copy(zer_hbm, sp)

        plsc.subcore_barrier()
        base = sid * ept

        def body(j, carry):
            off = pl.multiple_of(base + j * ch, ch)
            pltpu.sync_copy(idx_hbm.at[pl.ds(off, ch)], idx_v)
            pltpu.sync_copy(vals_hbm.at[pl.ds(off, ch), pl.ds(col0, cs)], val_v)
            pltpu.sync_copy(val_v, sp.at[idx_v], add=True)
            return carry

        lax.fori_loop(0, nch, body, 0)
        plsc.subcore_barrier()
        r0 = pl.multiple_of(sid * 624, 8)
        pltpu.sync_copy(sp.at[pl.ds(r0, rpt)],
                        out_hbm.at[pl.ds(r0, rpt), pl.ds(col0, cs)])

    return sk


def _make_scatter_edges():
    ept = E_PAD // NW
    ch = 128
    nch = ept // ch
    rpt = 640

    @functools.partial(
        pl.kernel, mesh=_MESH,
        out_type=jax.ShapeDtypeStruct((2 * N, 16), F32),
        scratch_types=[pltpu.VMEM((ch,), jnp.int32),
                       pltpu.VMEM((ch, 16), F32),
                       pltpu.VMEM_SHARED((N, 16), F32)],
    )
    def sk(idx_hbm, vals_hbm, zer_hbm, out_hbm, idx_v, val_v, sp):
        cid = lax.axis_index("c")
        sid = lax.axis_index("s")

        @pl.when(sid == 0)
        def _():
            pltpu.sync_copy(zer_hbm, sp)

        plsc.subcore_barrier()
        base = (cid * 16 + sid) * ept

        def body(j, carry):
            off = pl.multiple_of(base + j * ch, ch)
            pltpu.sync_copy(idx_hbm.at[pl.ds(off, ch)], idx_v)
            pltpu.sync_copy(vals_hbm.at[pl.ds(off, ch)], val_v)
            pltpu.sync_copy(val_v, sp.at[idx_v], add=True)
            return carry

        lax.fori_loop(0, nch, body, 0)
        plsc.subcore_barrier()
        r0 = pl.multiple_of(sid * 624, 8)
        pltpu.sync_copy(sp.at[pl.ds(r0, rpt)],
                        out_hbm.at[pl.ds(cid * N + r0, rpt)])

    return sk


_gather_gat = _make_gather2(384, 128)
_gather_mlp = _make_gather2(128, 128)
_gather_nn = _make_gather1(128)
_scatter_cols = _make_scatter_cols()
_scatter_edges = _make_scatter_edges()


def _row_mask(be):
    gid = pl.program_id(0) * be + lax.broadcasted_iota(jnp.int32, (be, 1), 0)
    return (gid < E).astype(F32)


def _nnconv_pre_body(x_ref, w2m_ref, rootw_ref, t2_ref, root_ref):
    x = x_ref[...]
    t2_ref[...] = jnp.dot(x, w2m_ref[...], preferred_element_type=F32)
    root_ref[...] = jnp.dot(x, rootw_ref[...], preferred_element_type=F32)


def _nnconv_msg_body(gt_ref, ea_ref, w1p_ref, b1_ref, expk_ref, c64_ref, o_ref):
    h = jnp.maximum(jnp.dot(ea_ref[...], w1p_ref[...],
                            preferred_element_type=F32) + b1_ref[...], 0.0)
    h2e = jnp.dot(h, expk_ref[...], preferred_element_type=F32)
    gt = gt_ref[...]
    msg = jnp.dot(h2e * gt[:, :64], c64_ref[...],
                  preferred_element_type=F32) + gt[:, 64:72]
    o_ref[...] = jnp.concatenate(
        [msg * _row_mask(BE), jnp.zeros((BE, 8), F32)], axis=1)


def _nnconv_post_body(root_ref, p0_ref, p1_ref, b_ref, o_ref):
    v = root_ref[...] + p0_ref[...][:, :8] + p1_ref[...][:, :8] + b_ref[...]
    o_ref[...] = jnp.where(v > 0, v, jnp.exp(v) - 1.0)


def _gat_pre_body(xh_ref, lin_ref, msrc_ref, mdst_ref, xt_ref, dt_ref):
    xl = jnp.dot(xh_ref[...], lin_ref[...], preferred_element_type=F32)
    a_s = jnp.dot(xl, msrc_ref[...], preferred_element_type=F32)
    a_d = jnp.dot(xl, mdst_ref[...], preferred_element_type=F32)
    t = a_s + a_d
    b = jnp.where(t >= 0, t, NEG * t)
    xt_ref[...] = jnp.concatenate([xl, a_s, jnp.zeros((BN, 120), F32)], axis=1)
    dt_ref[...] = jnp.concatenate([a_d, b, jnp.zeros((BN, 112), F32)], axis=1)


def _gat_mid_body(gx_ref, gd_ref, exp8_ref, wm_ref, ev_ref):
    gx = gx_ref[...]
    gd = gd_ref[...]
    t = gx[:, 256:264] + gd[:, 0:8]
    al = jnp.where(t >= 0, t, NEG * t)
    ev = jnp.exp(al - gd[:, 8:16]) * _row_mask(BE)
    evx = jnp.dot(ev, exp8_ref[...], preferred_element_type=F32)
    wm_ref[...] = gx[:, :256] * evx
    ev_ref[...] = jnp.concatenate([ev, jnp.zeros((BE, 8), F32)], axis=1)


def _gat_post_body(s_ref, xt_ref, p0_ref, p1_ref, prev_ref, b_ref, exp8_ref,
                   col_ref, o_ref):
    r = 1.0 / (p0_ref[...][:, :8] + p1_ref[...][:, :8] + 1.0)
    rexp = jnp.dot(r, exp8_ref[...], preferred_element_type=F32)
    tmp = (s_ref[...] + xt_ref[...][:, :256]) * rexp
    outm = jnp.dot(tmp, col_ref[...], preferred_element_type=F32) + b_ref[...]
    v = outm + prev_ref[...]
    o_ref[...] = jnp.where(v > 0, v, jnp.exp(v) - 1.0)


def _mlp_body(g1_ref, g2_ref, w0a_ref, w0b_ref, b0_ref, w1_ref, b1_ref,
              w2_ref, b2_ref, o_ref):
    h0 = jnp.maximum(
        jnp.dot(g1_ref[...][:, :HID], w0a_ref[...], preferred_element_type=F32)
        + jnp.dot(g2_ref[...][:, :HID], w0b_ref[...], preferred_element_type=F32)
        + b0_ref[...], 0.0)
    h1 = jnp.maximum(
        jnp.dot(h0, w1_ref[...], preferred_element_type=F32) + b1_ref[...] + h0,
        0.0)
    o_ref[...] = jnp.dot(h1, w2_ref[...], preferred_element_type=F32) + b2_ref[...]


def _full(c):
    return pl.BlockSpec(c, lambda i: (0,) * len(c))


def _rows(b, c):
    return pl.BlockSpec((b, c), lambda i: (i, 0))


def _nnconv_pre(x, w2m, rootw):
    return pl.pallas_call(
        _nnconv_pre_body, grid=(N // BN,),
        in_specs=[_rows(BN, IN), _full((IN, 128)), _full((IN, 8))],
        out_specs=(_rows(BN, 128), _rows(BN, 8)),
        out_shape=(jax.ShapeDtypeStruct((N, 128), F32),
                   jax.ShapeDtypeStruct((N, 8), F32)),
    )(x, w2m, rootw)


def _nnconv_msg(gt, ea8, w1p, b1, expk, c64):
    return pl.pallas_call(
        _nnconv_msg_body, grid=(E_PAD // BE,),
        in_specs=[_rows(BE, 128), _rows(BE, 8), _full((8, 8)), _full((1, 8)),
                  _full((8, 64)), _full((64, 8))],
        out_specs=_rows(BE, 16),
        out_shape=jax.ShapeDtypeStruct((E_PAD, 16), F32),
    )(gt, ea8, w1p, b1, expk, c64)


def _nnconv_post(root, p0, p1, b):
    return pl.pallas_call(
        _nnconv_post_body, grid=(N // BN,),
        in_specs=[_rows(BN, 8), _rows(BN, 16), _rows(BN, 16), _full((1, 8))],
        out_specs=_rows(BN, 8),
        out_shape=jax.ShapeDtypeStruct((N, 8), F32),
    )(root, p0, p1, b)


def _gat_pre(xh, lin, msrc, mdst):
    cin = xh.shape[1]
    return pl.pallas_call(
        _gat_pre_body, grid=(N // BN,),
        in_specs=[_rows(BN, cin), _full((cin, 256)), _full((256, 8)),
                  _full((256, 8))],
        out_specs=(_rows(BN, 384), _rows(BN, 128)),
        out_shape=(jax.ShapeDtypeStruct((N, 384), F32),
                   jax.ShapeDtypeStruct((N, 128), F32)),
    )(xh, lin, msrc, mdst)


def _gat_mid(gx, gd, exp8):
    return pl.pallas_call(
        _gat_mid_body, grid=(E_PAD // BE,),
        in_specs=[_rows(BE, 384), _rows(BE, 128), _full((8, 256))],
        out_specs=(_rows(BE, 256), _rows(BE, 16)),
        out_shape=(jax.ShapeDtypeStruct((E_PAD, 256), F32),
                   jax.ShapeDtypeStruct((E_PAD, 16), F32)),
    )(gx, gd, exp8)


def _gat_post(s, xt, p0, p1, prev, b, exp8, col):
    return pl.pallas_call(
        _gat_post_body, grid=(N // BN,),
        in_specs=[_rows(BN, 256), _rows(BN, 384), _rows(BN, 16), _rows(BN, 16),
                  _rows(BN, HID), _full((1, HID)), _full((8, 256)),
                  _full((256, HID))],
        out_specs=_rows(BN, HID),
        out_shape=jax.ShapeDtypeStruct((N, HID), F32),
    )(s, xt, p0, p1, prev, b, exp8, col)


def _edge_mlp(g1, g2, w0a, w0b, b0, w1, b1, w2p, b2p):
    return pl.pallas_call(
        _mlp_body, grid=(E_PAD // BE,),
        in_specs=[_rows(BE, 128), _rows(BE, 128), _full((HID, HID)),
                  _full((HID, HID)), _full((1, HID)), _full((HID, HID)),
                  _full((1, HID)), _full((HID, 8)), _full((1, 8))],
        out_specs=_rows(BE, 8),
        out_shape=jax.ShapeDtypeStruct((E_PAD, 8), F32),
    )(g1, g2, w0a, w0b, b0, w1, b1, w2p, b2p)


def kernel(x, edge_index, edge_attr, enn_w1, enn_b1, enn_w2, enn_b2, root_w,
           nnconv_b,
           gat0_lin, gat0_asrc, gat0_adst, gat0_b,
           gat1_lin, gat1_asrc, gat1_adst, gat1_b,
           gat2_lin, gat2_asrc, gat2_adst, gat2_b,
           gat3_lin, gat3_asrc, gat3_adst, gat3_b,
           mlp0_w, mlp0_b, mlp1_w, mlp1_b, mlp2_w, mlp2_b):
    src = edge_index[0]
    dst = edge_index[1]
    pad = jnp.zeros((E_PAD - E,), jnp.int32)
    srcp = jnp.concatenate([src, pad])
    dstp = jnp.concatenate([dst, pad])

    eye8 = jnp.eye(8, dtype=F32)
    exp8 = jnp.repeat(eye8, 32, axis=1)
    col = jnp.tile(jnp.eye(HID, dtype=F32), (8, 1)) / 8.0
    expk = jnp.repeat(eye8, 8, axis=1)
    c64 = jnp.tile(eye8, (8, 1))
    zer128 = jnp.zeros((N, 128), F32)
    zer16 = jnp.zeros((N, 16), F32)

    w2m = jnp.concatenate(
        [enn_w2.reshape(8, IN, EMB).transpose(1, 0, 2).reshape(IN, 64),
         enn_b2.reshape(IN, EMB), jnp.zeros((IN, 56), F32)], axis=1)
    t2p, xroot = _nnconv_pre(x, w2m, root_w)
    gt = _gather_nn(srcp, t2p)
    ea8 = jnp.pad(edge_attr, ((0, E_PAD - E), (0, 5)))
    w1p = jnp.pad(enn_w1, ((0, 5), (0, 0)))
    msgp = _nnconv_msg(gt, ea8, w1p, enn_b1[None, :], expk, c64)
    parts = _scatter_edges(dstp, msgp, zer16)
    xh = _nnconv_post(xroot, parts[:N], parts[N:], nnconv_b[None, :])

    prev = jnp.zeros((N, HID), F32)
    for i, (lin, asrc, adst, bias) in enumerate(
            ((gat0_lin, gat0_asrc, gat0_adst, gat0_b),
             (gat1_lin, gat1_asrc, gat1_adst, gat1_b),
             (gat2_lin, gat2_asrc, gat2_adst, gat2_b),
             (gat3_lin, gat3_asrc, gat3_adst, gat3_b))):
        msrc = (eye8[:, None, :] * asrc[:, :, None]).reshape(256, 8)
        mdst = (eye8[:, None, :] * adst[:, :, None]).reshape(256, 8)
        xt, dt = _gat_pre(xh, lin, msrc, mdst)
        gx, gd = _gather_gat(srcp, xt, dstp, dt)
        wm, evp = _gat_mid(gx, gd, exp8)
        sparts = _scatter_edges(dstp, evp, zer16)
        s = _scatter_cols(dstp, wm, zer128)
        xh = _gat_post(s, xt, sparts[:N], sparts[N:], prev, bias[None, :],
                       exp8, col)
        prev = xh

    xhp = jnp.pad(xh, ((0, 0), (0, 128 - HID)))
    g1, g2 = _gather_mlp(srcp, xhp, dstp, xhp)
    w2p = jnp.zeros((HID, 8), F32).at[:, :1].set(mlp2_w)
    b2p = jnp.zeros((1, 8), F32).at[:, :1].set(mlp2_b[None, :])
    outp = _edge_mlp(g1, g2, mlp0_w[:HID], mlp0_w[HID:], mlp0_b[None, :],
                     mlp1_w, mlp1_b[None, :], w2p, b2p)
    return outp[:E, :1]


# speedup vs baseline: 22.1612x; 1.5201x over previous
import functools
import jax
import jax.numpy as jnp
from jax import lax
from jax.experimental import pallas as pl
from jax.experimental.pallas import tpu as pltpu
from jax.experimental.pallas import tpu_sc as plsc

N = 10000
E = 160000
E_PAD = 163840
IN = 128
EMB = 8
H = 8
HID = 32
NEG = 0.2

NW = 32
BE = 1024
BN = 2000
F32 = jnp.float32

_MESH = plsc.VectorSubcoreMesh(core_axis_name="c", subcore_axis_name="s")


def _make_gather2(ca, cb):
    epw = E_PAD // NW
    ch = 128
    nch = epw // ch

    @functools.partial(
        pl.kernel, mesh=_MESH,
        out_type=(jax.ShapeDtypeStruct((E_PAD, ca), F32),
                  jax.ShapeDtypeStruct((E_PAD, cb), F32)),
        scratch_types=[pltpu.VMEM((ch,), jnp.int32), pltpu.VMEM((ch,), jnp.int32),
                       pltpu.VMEM((ch, ca), F32), pltpu.VMEM((ch, cb), F32),
                       pltpu.SemaphoreType.DMA, pltpu.SemaphoreType.DMA],
    )
    def gk(ia_hbm, ta_hbm, ib_hbm, tb_hbm, oa_hbm, ob_hbm,
           ia_v, ib_v, ra_v, rb_v, sa, sb):
        wid = lax.axis_index("s") * 2 + lax.axis_index("c")
        base = wid * epw

        def body(j, carry):
            off = pl.multiple_of(base + j * ch, ch)
            pltpu.sync_copy(ia_hbm.at[pl.ds(off, ch)], ia_v)
            pltpu.sync_copy(ib_hbm.at[pl.ds(off, ch)], ib_v)
            da = pltpu.async_copy(ta_hbm.at[ia_v], ra_v, sa)
            db = pltpu.async_copy(tb_hbm.at[ib_v], rb_v, sb)
            da.wait()
            db.wait()
            pltpu.sync_copy(ra_v, oa_hbm.at[pl.ds(off, ch)])
            pltpu.sync_copy(rb_v, ob_hbm.at[pl.ds(off, ch)])
            return carry

        lax.fori_loop(0, nch, body, 0)

    return gk


def _make_gather1(ca):
    epw = E_PAD // NW
    ch = 128
    nch = epw // ch

    @functools.partial(
        pl.kernel, mesh=_MESH,
        out_type=jax.ShapeDtypeStruct((E_PAD, ca), F32),
        scratch_types=[pltpu.VMEM((ch,), jnp.int32),
                       pltpu.VMEM((ch, ca), F32),
                       pltpu.SemaphoreType.DMA],
    )
    def gk(ia_hbm, ta_hbm, oa_hbm, ia_v, ra_v, sa):
        wid = lax.axis_index("s") * 2 + lax.axis_index("c")
        base = wid * epw

        def body(j, carry):
            off = pl.multiple_of(base + j * ch, ch)
            pltpu.sync_copy(ia_hbm.at[pl.ds(off, ch)], ia_v)
            pltpu.async_copy(ta_hbm.at[ia_v], ra_v, sa).wait()
            pltpu.sync_copy(ra_v, oa_hbm.at[pl.ds(off, ch)])
            return carry

        lax.fori_loop(0, nch, body, 0)

    return gk


def _make_scatter_cols():
    ept = E_PAD // 16
    ch = 128
    nch = ept // ch
    cs = 128
    rpt = 640

    @functools.partial(
        pl.kernel, mesh=_MESH,
        out_type=jax.ShapeDtypeStruct((N, 256), F32),
        scratch_types=[pltpu.VMEM((ch,), jnp.int32),
                       pltpu.VMEM((ch, cs), F32),
                       pltpu.VMEM_SHARED((N, cs), F32)],
    )
    def sk(idx_hbm, vals_hbm, zer_hbm, out_hbm, idx_v, val_v, sp):
        cid = lax.axis_index("c")
        sid = lax.axis_index("s")
        col0 = cid * cs

        @pl.when(sid == 0)
        def _():
            pltpu.sync_copy(zer_hbm, sp)

        plsc.subcore_barrier()
        base = sid * ept

        def body(j, carry):
            off = pl.multiple_of(base + j * ch, ch)
            pltpu.sync_copy(idx_hbm.at[pl.ds(off, ch)], idx_v)
            pltpu.sync_copy(vals_hbm.at[pl.ds(off, ch), pl.ds(col0, cs)], val_v)
            pltpu.sync_copy(val_v, sp.at[idx_v], add=True)
            return carry

        lax.fori_loop(0, nch, body, 0)
        plsc.subcore_barrier()
        r0 = pl.multiple_of(sid * 624, 8)
        pltpu.sync_copy(sp.at[pl.ds(r0, rpt)],
                        out_hbm.at[pl.ds(r0, rpt), pl.ds(col0, cs)])

    return sk


def _make_scatter_edges():
    ept = E_PAD // NW
    ch = 128
    nch = ept // ch
    rpt = 640

    @functools.partial(
        pl.kernel, mesh=_MESH,
        out_type=jax.ShapeDtypeStruct((2 * N, 16), F32),
        scratch_types=[pltpu.VMEM((ch,), jnp.int32),
                       pltpu.VMEM((ch, 16), F32),
                       pltpu.VMEM_SHARED((N, 16), F32)],
    )
    def sk(idx_hbm, vals_hbm, zer_hbm, out_hbm, idx_v, val_v, sp):
        cid = lax.axis_index("c")
        sid = lax.axis_index("s")

        @pl.when(sid == 0)
        def _():
            pltpu.sync_copy(zer_hbm, sp)

        plsc.subcore_barrier()
        base = (cid * 16 + sid) * ept

        def body(j, carry):
            off = pl.multiple_of(base + j * ch, ch)
            pltpu.sync_copy(idx_hbm.at[pl.ds(off, ch)], idx_v)
            pltpu.sync_copy(vals_hbm.at[pl.ds(off, ch)], val_v)
            pltpu.sync_copy(val_v, sp.at[idx_v], add=True)
            return carry

        lax.fori_loop(0, nch, body, 0)
        plsc.subcore_barrier()
        r0 = pl.multiple_of(sid * 624, 8)
        pltpu.sync_copy(sp.at[pl.ds(r0, rpt)],
                        out_hbm.at[pl.ds(cid * N + r0, rpt)])

    return sk


_gather_gat = _make_gather2(256, 128)
_gather_mlp = _make_gather2(128, 128)
_gather_nn = _make_gather1(128)
_scatter_cols = _make_scatter_cols()
_scatter_edges = _make_scatter_edges()


def _row_mask(be):
    gid = pl.program_id(0) * be + lax.broadcasted_iota(jnp.int32, (be, 1), 0)
    return (gid < E).astype(F32)


def _nnconv_pre_body(x_ref, w2m_ref, rootw_ref, t2_ref, root_ref):
    x = x_ref[...]
    t2_ref[...] = jnp.dot(x, w2m_ref[...], preferred_element_type=F32)
    root_ref[...] = jnp.dot(x, rootw_ref[...], preferred_element_type=F32)


def _nnconv_msg_body(gt_ref, ea_ref, w1p_ref, b1_ref, expk_ref, c64_ref, o_ref):
    h = jnp.maximum(jnp.dot(ea_ref[...], w1p_ref[...],
                            preferred_element_type=F32) + b1_ref[...], 0.0)
    h2e = jnp.dot(h, expk_ref[...], preferred_element_type=F32)
    gt = gt_ref[...]
    msg = jnp.dot(h2e * gt[:, :64], c64_ref[...],
                  preferred_element_type=F32) + gt[:, 64:72]
    o_ref[...] = jnp.concatenate(
        [msg * _row_mask(BE), jnp.zeros((BE, 8), F32)], axis=1)


def _nnconv_post_body(root_ref, p0_ref, p1_ref, b_ref, o_ref):
    v = root_ref[...] + p0_ref[...][:, :8] + p1_ref[...][:, :8] + b_ref[...]
    o_ref[...] = jnp.where(v > 0, v, jnp.exp(v) - 1.0)


def _gat_pre_body(xh_ref, lin_ref, msrc_ref, mdst_ref, xt_ref, dt_ref):
    xl = jnp.dot(xh_ref[...], lin_ref[...], preferred_element_type=F32)
    a_s = jnp.dot(xl, msrc_ref[...], preferred_element_type=F32)
    a_d = jnp.dot(xl, mdst_ref[...], preferred_element_type=F32)
    t = a_s + a_d
    b = jnp.where(t >= 0, t, NEG * t)
    xt_ref[...] = xl
    dt_ref[...] = jnp.concatenate([a_d, b, jnp.zeros((BN, 112), F32)], axis=1)


def _gat_mid_body(gx_ref, gd_ref, msrc_ref, exp8_ref, wm_ref, ev_ref):
    gx = gx_ref[...]
    gd = gd_ref[...]
    a_s = jnp.dot(gx, msrc_ref[...], preferred_element_type=F32)
    t = a_s + gd[:, 0:8]
    al = jnp.where(t >= 0, t, NEG * t)
    ev = jnp.exp(al - gd[:, 8:16]) * _row_mask(BE)
    evx = jnp.dot(ev, exp8_ref[...], preferred_element_type=F32)
    wm_ref[...] = gx * evx
    ev_ref[...] = jnp.concatenate([ev, jnp.zeros((BE, 8), F32)], axis=1)


def _gat_post_body(s_ref, xt_ref, p0_ref, p1_ref, prev_ref, b_ref, exp8_ref,
                   col_ref, o_ref):
    r = 1.0 / (p0_ref[...][:, :8] + p1_ref[...][:, :8] + 1.0)
    rexp = jnp.dot(r, exp8_ref[...], preferred_element_type=F32)
    tmp = (s_ref[...] + xt_ref[...]) * rexp
    outm = jnp.dot(tmp, col_ref[...], preferred_element_type=F32) + b_ref[...]
    v = outm + prev_ref[...]
    o_ref[...] = jnp.where(v > 0, v, jnp.exp(v) - 1.0)


def _mlp_body(g1_ref, g2_ref, w0a_ref, w0b_ref, b0_ref, w1_ref, b1_ref,
              w2_ref, b2_ref, o_ref):
    h0 = jnp.maximum(
        jnp.dot(g1_ref[...][:, :HID], w0a_ref[...], preferred_element_type=F32)
        + jnp.dot(g2_ref[...][:, :HID], w0b_ref[...], preferred_element_type=F32)
        + b0_ref[...], 0.0)
    h1 = jnp.maximum(
        jnp.dot(h0, w1_ref[...], preferred_element_type=F32) + b1_ref[...] + h0,
        0.0)
    o_ref[...] = jnp.dot(h1, w2_ref[...], preferred_element_type=F32) + b2_ref[...]


def _full(c):
    return pl.BlockSpec(c, lambda i: (0,) * len(c))


def _rows(b, c):
    return pl.BlockSpec((b, c), lambda i: (i, 0))


def _nnconv_pre(x, w2m, rootw):
    return pl.pallas_call(
        _nnconv_pre_body, grid=(N // BN,),
        in_specs=[_rows(BN, IN), _full((IN, 128)), _full((IN, 8))],
        out_specs=(_rows(BN, 128), _rows(BN, 8)),
        out_shape=(jax.ShapeDtypeStruct((N, 128), F32),
                   jax.ShapeDtypeStruct((N, 8), F32)),
    )(x, w2m, rootw)


def _nnconv_msg(gt, ea8, w1p, b1, expk, c64):
    return pl.pallas_call(
        _nnconv_msg_body, grid=(E_PAD // BE,),
        in_specs=[_rows(BE, 128), _rows(BE, 8), _full((8, 8)), _full((1, 8)),
                  _full((8, 64)), _full((64, 8))],
        out_specs=_rows(BE, 16),
        out_shape=jax.ShapeDtypeStruct((E_PAD, 16), F32),
    )(gt, ea8, w1p, b1, expk, c64)


def _nnconv_post(root, p0, p1, b):
    return pl.pallas_call(
        _nnconv_post_body, grid=(N // BN,),
        in_specs=[_rows(BN, 8), _rows(BN, 16), _rows(BN, 16), _full((1, 8))],
        out_specs=_rows(BN, 8),
        out_shape=jax.ShapeDtypeStruct((N, 8), F32),
    )(root, p0, p1, b)


def _gat_pre(xh, lin, msrc, mdst):
    cin = xh.shape[1]
    return pl.pallas_call(
        _gat_pre_body, grid=(N // BN,),
        in_specs=[_rows(BN, cin), _full((cin, 256)), _full((256, 8)),
                  _full((256, 8))],
        out_specs=(_rows(BN, 256), _rows(BN, 128)),
        out_shape=(jax.ShapeDtypeStruct((N, 256), F32),
                   jax.ShapeDtypeStruct((N, 128), F32)),
    )(xh, lin, msrc, mdst)


def _gat_mid(gx, gd, msrc, exp8):
    return pl.pallas_call(
        _gat_mid_body, grid=(E_PAD // BE,),
        in_specs=[_rows(BE, 256), _rows(BE, 128), _full((256, 8)),
                  _full((8, 256))],
        out_specs=(_rows(BE, 256), _rows(BE, 16)),
        out_shape=(jax.ShapeDtypeStruct((E_PAD, 256), F32),
                   jax.ShapeDtypeStruct((E_PAD, 16), F32)),
    )(gx, gd, msrc, exp8)


def _gat_post(s, xt, p0, p1, prev, b, exp8, col):
    return pl.pallas_call(
        _gat_post_body, grid=(N // BN,),
        in_specs=[_rows(BN, 256), _rows(BN, 256), _rows(BN, 16), _rows(BN, 16),
                  _rows(BN, HID), _full((1, HID)), _full((8, 256)),
                  _full((256, HID))],
        out_specs=_rows(BN, HID),
        out_shape=jax.ShapeDtypeStruct((N, HID), F32),
    )(s, xt, p0, p1, prev, b, exp8, col)


def _edge_mlp(g1, g2, w0a, w0b, b0, w1, b1, w2p, b2p):
    return pl.pallas_call(
        _mlp_body, grid=(E_PAD // BE,),
        in_specs=[_rows(BE, 128), _rows(BE, 128), _full((HID, HID)),
                  _full((HID, HID)), _full((1, HID)), _full((HID, HID)),
                  _full((1, HID)), _full((HID, 8)), _full((1, 8))],
        out_specs=_rows(BE, 8),
        out_shape=jax.ShapeDtypeStruct((E_PAD, 8), F32),
    )(g1, g2, w0a, w0b, b0, w1, b1, w2p, b2p)


def kernel(x, edge_index, edge_attr, enn_w1, enn_b1, enn_w2, enn_b2, root_w,
           nnconv_b,
           gat0_lin, gat0_asrc, gat0_adst, gat0_b,
           gat1_lin, gat1_asrc, gat1_adst, gat1_b,
           gat2_lin, gat2_asrc, gat2_adst, gat2_b,
           gat3_lin, gat3_asrc, gat3_adst, gat3_b,
           mlp0_w, mlp0_b, mlp1_w, mlp1_b, mlp2_w, mlp2_b):
    src = edge_index[0]
    dst = edge_index[1]
    pad = (jnp.arange(E_PAD - E, dtype=jnp.int32) * 131) % N
    srcp = jnp.concatenate([src, pad])
    dstp = jnp.concatenate([dst, pad])

    eye8 = jnp.eye(8, dtype=F32)
    exp8 = jnp.repeat(eye8, 32, axis=1)
    col = jnp.tile(jnp.eye(HID, dtype=F32), (8, 1)) / 8.0
    expk = jnp.repeat(eye8, 8, axis=1)
    c64 = jnp.tile(eye8, (8, 1))
    zer128 = jnp.zeros((N, 128), F32)
    zer16 = jnp.zeros((N, 16), F32)

    w2m = jnp.concatenate(
        [enn_w2.reshape(8, IN, EMB).transpose(1, 0, 2).reshape(IN, 64),
         enn_b2.reshape(IN, EMB), jnp.zeros((IN, 56), F32)], axis=1)
    t2p, xroot = _nnconv_pre(x, w2m, root_w)
    gt = _gather_nn(srcp, t2p)
    ea8 = jnp.pad(edge_attr, ((0, E_PAD - E), (0, 5)))
    w1p = jnp.pad(enn_w1, ((0, 5), (0, 0)))
    msgp = _nnconv_msg(gt, ea8, w1p, enn_b1[None, :], expk, c64)
    parts = _scatter_edges(dstp, msgp, zer16)
    xh = _nnconv_post(xroot, parts[:N], parts[N:], nnconv_b[None, :])

    prev = jnp.zeros((N, HID), F32)
    for i, (lin, asrc, adst, bias) in enumerate(
            ((gat0_lin, gat0_asrc, gat0_adst, gat0_b),
             (gat1_lin, gat1_asrc, gat1_adst, gat1_b),
             (gat2_lin, gat2_asrc, gat2_adst, gat2_b),
             (gat3_lin, gat3_asrc, gat3_adst, gat3_b))):
        msrc = (eye8[:, None, :] * asrc[:, :, None]).reshape(256, 8)
        mdst = (eye8[:, None, :] * adst[:, :, None]).reshape(256, 8)
        xt, dt = _gat_pre(xh, lin, msrc, mdst)
        gx, gd = _gather_gat(srcp, xt, dstp, dt)
        wm, evp = _gat_mid(gx, gd, msrc, exp8)
        sparts = _scatter_edges(dstp, evp, zer16)
        s = _scatter_cols(dstp, wm, zer128)
        xh = _gat_post(s, xt, sparts[:N], sparts[N:], prev, bias[None, :],
                       exp8, col)
        prev = xh

    xhp = jnp.pad(xh, ((0, 0), (0, 128 - HID)))
    g1, g2 = _gather_mlp(srcp, xhp, dstp, xhp)
    w2p = jnp.zeros((HID, 8), F32).at[:, :1].set(mlp2_w)
    b2p = jnp.zeros((1, 8), F32).at[:, :1].set(mlp2_b[None, :])
    outp = _edge_mlp(g1, g2, mlp0_w[:HID], mlp0_w[HID:], mlp0_b[None, :],
                     mlp1_w, mlp1_b[None, :], w2p, b2p)
    return outp[:E, :1]
